# baseline (speedup 1.0000x reference)
import functools

import jax
import jax.numpy as jnp
from jax import lax
from jax.experimental import pallas as pl
from jax.experimental.pallas import tpu as pltpu
from jax.experimental.pallas import tpu_sc as plsc

B, N, K = 8, 2048, 20
M = B * N
T = M * K
_NW = 32
_CH = 128

bf16 = jnp.bfloat16
f32 = jnp.float32


def _leaky(x):
    return jnp.where(x >= 0, x, 0.2 * x)


def _knn_body(R, xt_ref, xc_ref, idx_ref, vscr):
    xr = xt_ref[0]
    xa = xc_ref[0]
    mm = jnp.dot(xr.astype(bf16), xa.astype(bf16),
                 preferred_element_type=f32)
    xxr = jnp.sum(xr * xr, axis=1, keepdims=True)
    xxa = jnp.sum(xa * xa, axis=0, keepdims=True)
    vscr[...] = (-xxr - (-2.0 * mm)) - xxa
    base = pl.program_id(0) * N
    iota = lax.broadcasted_iota(jnp.int32, (R, N), 1)
    kiota = lax.broadcasted_iota(jnp.int32, (R, 32), 1)

    def body(t, acc):
        v = vscr[...]
        m = jnp.max(v, axis=1, keepdims=True)
        widx = jnp.min(jnp.where(v == m, iota, N), axis=1, keepdims=True)
        vscr[...] = jnp.where(iota == widx, -jnp.inf, v)
        return jnp.where(kiota == t, widx, acc)

    acc = lax.fori_loop(0, K, body, jnp.zeros((R, 32), jnp.int32), unroll=20)
    idx_ref[0] = acc[:, :K] + base


def _knn(xt, xc, R=1024):
    Bk, _, C = xt.shape
    return pl.pallas_call(
        functools.partial(_knn_body, R),
        grid=(Bk, N // R),
        in_specs=[
            pl.BlockSpec((1, R, C), lambda b, r: (b, r, 0)),
            pl.BlockSpec((1, C, N), lambda b, r: (b, 0, 0)),
        ],
        out_specs=pl.BlockSpec((1, R, K), lambda b, r: (b, r, 0)),
        out_shape=jax.ShapeDtypeStruct((Bk, N, K), jnp.int32),
        scratch_shapes=[pltpu.VMEM((R, N), f32)],
    )(xt, xc)


def _sc_gather(table, idxflat):
    D = table.shape[1]
    Tk = idxflat.shape[0]
    per_w = Tk // _NW
    n_ch = per_w // _CH
    mesh = plsc.VectorSubcoreMesh(core_axis_name="c", subcore_axis_name="s")

    NB = 4

    @functools.partial(
        pl.kernel,
        out_type=jax.ShapeDtypeStruct((Tk, D), f32),
        mesh=mesh,
        compiler_params=pltpu.CompilerParams(use_tc_tiling_on_sc=False),
        scratch_types=[
            pltpu.VMEM((n_ch, _CH), jnp.int32),
            [pltpu.VMEM((_CH, D), f32) for _ in range(NB)],
            [pltpu.SemaphoreType.DMA for _ in range(NB)],
        ],
    )
    def gk(table_hbm, idx_hbm, out_hbm, idx_v, rows_v, sems):
        wid = lax.axis_index("s") * 2 + lax.axis_index("c")
        base = wid * per_w
        pltpu.sync_copy(idx_hbm.at[wid], idx_v)

        def body(g, c):
            i0 = g * NB
            copies = []
            for b in range(NB):
                copies.append(pltpu.async_copy(
                    table_hbm.at[idx_v.at[i0 + b]], rows_v[b], sems[b]))
            for b in range(NB):
                copies[b].wait()
                pltpu.sync_copy(rows_v[b],
                                out_hbm.at[pl.ds((base + (i0 + b) * _CH), _CH)])
            return c

        lax.fori_loop(0, n_ch // NB, body, 0)

    return gk(table, idxflat.reshape(_NW, n_ch, _CH))


def _edge_body(R, C, two_conv, g_ref, c_ref, w1_ref, ga1_ref, ba1_ref,
               w2_ref, ga2_ref, ba2_ref, out_ref):
    g = g_ref[0].reshape(R, K, C)
    c = c_ref[0]
    cb = jnp.broadcast_to(c[:, None, :], (R, K, C))
    fb = jnp.concatenate([(g - cb).astype(bf16), cb.astype(bf16)], axis=2)
    h = jnp.dot(fb.reshape(R * K, 2 * C), w1_ref[...],
                preferred_element_type=f32)
    h = _leaky(h * ga1_ref[0] + ba1_ref[0])
    if two_conv:
        h = jnp.dot(h.astype(bf16), w2_ref[...], preferred_element_type=f32)
        h = _leaky(h * ga2_ref[0] + ba2_ref[0])
    out_ref[0] = jnp.max(h.reshape(R, K, 64), axis=1)


def _edge(G, xt, w1, ga1, ba1, w2, ga2, ba2, two_conv, R=128):
    Bk, _, C = xt.shape
    return pl.pallas_call(
        functools.partial(_edge_body, R, C, two_conv),
        grid=(Bk, N // R),
        in_specs=[
            pl.BlockSpec((1, R * K, C), lambda b, r: (b, r, 0)),
            pl.BlockSpec((1, R, C), lambda b, r: (b, r, 0)),
            pl.BlockSpec((2 * C, 64), lambda b, r: (0, 0)),
            pl.BlockSpec((1, 64), lambda b, r: (0, 0)),
            pl.BlockSpec((1, 64), lambda b, r: (0, 0)),
            pl.BlockSpec((64, 64), lambda b, r: (0, 0)),
            pl.BlockSpec((1, 64), lambda b, r: (0, 0)),
            pl.BlockSpec((1, 64), lambda b, r: (0, 0)),
        ],
        out_specs=pl.BlockSpec((1, R, 64), lambda b, r: (b, r, 0)),
        out_shape=jax.ShapeDtypeStruct((Bk, N, 64), f32),
    )(G, xt, w1, ga1, ba1, w2, ga2, ba2)


def _tail_body(x1_ref, x2_ref, x3_ref, w6_ref, w7h_ref, w7x_ref, w8_ref,
               wc_ref, ga6_ref, ba6_ref, ga7_ref, ba7_ref, ga8_ref, ba8_ref,
               out_ref):
    xc = jnp.concatenate([x1_ref[0], x2_ref[0], x3_ref[0]], axis=1)
    xcb = xc.astype(bf16)
    h6 = _leaky(jnp.dot(xcb, w6_ref[...], preferred_element_type=f32)
                * ga6_ref[0] + ba6_ref[0])
    hmax = jnp.max(h6, axis=0, keepdims=True)
    t7 = jnp.dot(hmax.astype(bf16), w7h_ref[...], preferred_element_type=f32)
    h7 = _leaky((jnp.dot(xcb, w7x_ref[...], preferred_element_type=f32)
                 + t7) * ga7_ref[0] + ba7_ref[0])
    h8 = _leaky(jnp.dot(h7.astype(bf16), w8_ref[...],
                        preferred_element_type=f32)
                * ga8_ref[0] + ba8_ref[0])
    out_ref[0] = jnp.dot(h8.astype(bf16), wc_ref[...],
                         preferred_element_type=f32)


def _tail(x1t, x2t, x3t, w6, w7h, w7x, w8, wc, ga6, ba6, ga7, ba7, ga8, ba8):
    Bk = x1t.shape[0]
    cfull = lambda shape: pl.BlockSpec(shape, lambda b: tuple(0 for _ in shape))
    return pl.pallas_call(
        _tail_body,
        grid=(Bk,),
        in_specs=[
            pl.BlockSpec((1, N, 64), lambda b: (b, 0, 0)),
            pl.BlockSpec((1, N, 64), lambda b: (b, 0, 0)),
            pl.BlockSpec((1, N, 64), lambda b: (b, 0, 0)),
            cfull((192, 1024)),
            cfull((1024, 512)),
            cfull((192, 512)),
            cfull((512, 256)),
            cfull((256, 17)),
            cfull((1, 1024)), cfull((1, 1024)),
            cfull((1, 512)), cfull((1, 512)),
            cfull((1, 256)), cfull((1, 256)),
        ],
        out_specs=pl.BlockSpec((1, N, 17), lambda b: (b, 0, 0)),
        out_shape=jax.ShapeDtypeStruct((Bk, N, 17), f32),
    )(x1t, x2t, x3t, w6, w7h, w7x, w8, wc, ga6, ba6, ga7, ba7, ga8, ba8)


def kernel(x_in, W1, W2, W3, W4, W5, W6, W7, W8, Wcls,
           g1, b1, g2, b2, g3, b3, g4, b4, g5, b5, g6, b6, g7, b7, g8, b8):
    x = x_in[0].astype(f32)
    inv = 1.0 / jnp.sqrt(jnp.asarray(1.0 + 1e-5, f32))

    def bnp(g, b):
        return (g * inv).reshape(1, -1), b.reshape(1, -1)

    ga1, ba1 = bnp(g1, b1)
    ga2, ba2 = bnp(g2, b2)
    ga3, ba3 = bnp(g3, b3)
    ga4, ba4 = bnp(g4, b4)
    ga5, ba5 = bnp(g5, b5)
    ga6, ba6 = bnp(g6, b6)
    ga7, ba7 = bnp(g7, b7)
    ga8, ba8 = bnp(g8, b8)

    def edge_w1(W, C):
        Ctrue = W.shape[1] // 2
        w = jnp.zeros((2 * C, 64), f32)
        w = w.at[:Ctrue].set(W[:, :Ctrue].T)
        w = w.at[C:C + Ctrue].set(W[:, Ctrue:].T)
        return w.astype(bf16)

    w1e = edge_w1(W1, 16)
    w2e = W2.T.astype(bf16)
    w3e = edge_w1(W3, 64)
    w4e = W4.T.astype(bf16)
    w5e = edge_w1(W5, 64)
    w6e = W6.T.astype(bf16)
    w7he = W7[:, :1024].T.astype(bf16)
    w7xe = W7[:, 1024:].T.astype(bf16)
    w8e = W8.T.astype(bf16)
    wce = Wcls.T.astype(bf16)

    def half(xh):
        Bk = xh.shape[0]
        Mk, Tk = Bk * N, Bk * N * K
        xpad = jnp.pad(xh, ((0, 0), (0, 10), (0, 0)))
        xt = jnp.transpose(xpad, (0, 2, 1))
        idx1 = _knn(xt, xpad)
        G1 = _sc_gather(xt.reshape(Mk, 16), idx1.reshape(Tk))
        x1t = _edge(G1.reshape(Bk, N * K, 16), xt, w1e, ga1, ba1,
                    w2e, ga2, ba2, True)
        x1c = jnp.transpose(x1t, (0, 2, 1))
        idx2 = _knn(x1t, x1c)
        G2 = _sc_gather(x1t.reshape(Mk, 64), idx2.reshape(Tk))
        x2t = _edge(G2.reshape(Bk, N * K, 64), x1t, w3e, ga3, ba3,
                    w4e, ga4, ba4, True)
        x2c = jnp.transpose(x2t, (0, 2, 1))
        idx3 = _knn(x2t, x2c)
        G3 = _sc_gather(x2t.reshape(Mk, 64), idx3.reshape(Tk))
        x3t = _edge(G3.reshape(Bk, N * K, 64), x2t, w5e, ga5, ba5,
                    w4e, ga5, ba5, False)
        return _tail(x1t, x2t, x3t, w6e, w7he, w7xe, w8e, wce,
                     ga6, ba6, ga7, ba7, ga8, ba8)

    out = jnp.concatenate([half(x[:4]), half(x[4:])], axis=0)
    return jnp.transpose(out, (0, 2, 1))


# speedup vs baseline: 1.2055x; 1.2055x over previous
import functools

import jax
import jax.numpy as jnp
from jax import lax
from jax.experimental import pallas as pl
from jax.experimental.pallas import tpu as pltpu
from jax.experimental.pallas import tpu_sc as plsc

B, N, K = 8, 2048, 20
M = B * N
T = M * K
_NW = 32
_CH = 128

bf16 = jnp.bfloat16
f32 = jnp.float32


def _leaky(x):
    return jnp.where(x >= 0, x, 0.2 * x)


def _knn_body(R, xt_ref, xc_ref, idx_ref, vscr):
    xr = xt_ref[0]
    xa = xc_ref[0]
    mm = jnp.dot(xr.astype(bf16), xa.astype(bf16),
                 preferred_element_type=f32)
    xxr = jnp.sum(xr * xr, axis=1, keepdims=True)
    xxa = jnp.sum(xa * xa, axis=0, keepdims=True)
    vscr[...] = (-xxr - (-2.0 * mm)) - xxa
    base = pl.program_id(0) * N
    iota = lax.broadcasted_iota(jnp.int32, (R, N), 1)
    kiota = lax.broadcasted_iota(jnp.int32, (R, 32), 1)

    def body(t, acc):
        v = vscr[...]
        m = jnp.max(v, axis=1, keepdims=True)
        widx = jnp.min(jnp.where(v == m, iota, N), axis=1, keepdims=True)
        vscr[...] = jnp.where(iota == widx, -jnp.inf, v)
        return jnp.where(kiota == t, widx, acc)

    acc = lax.fori_loop(0, K, body, jnp.zeros((R, 32), jnp.int32), unroll=10)
    idx_ref[0] = acc[:, :K] + base


def _knn(xt, xc, R=1024):
    Bk, _, C = xt.shape
    return pl.pallas_call(
        functools.partial(_knn_body, R),
        grid=(Bk, N // R),
        in_specs=[
            pl.BlockSpec((1, R, C), lambda b, r: (b, r, 0)),
            pl.BlockSpec((1, C, N), lambda b, r: (b, 0, 0)),
        ],
        out_specs=pl.BlockSpec((1, R, K), lambda b, r: (b, r, 0)),
        out_shape=jax.ShapeDtypeStruct((Bk, N, K), jnp.int32),
        scratch_shapes=[pltpu.VMEM((R, N), f32)],
    )(xt, xc)


def _sc_gather(table, idxflat):
    D = table.shape[1]
    Tk = idxflat.shape[0]
    per_w = Tk // _NW
    n_ch = per_w // _CH
    mesh = plsc.VectorSubcoreMesh(core_axis_name="c", subcore_axis_name="s")

    NB = 4

    @functools.partial(
        pl.kernel,
        out_type=jax.ShapeDtypeStruct((Tk, D), f32),
        mesh=mesh,
        compiler_params=pltpu.CompilerParams(use_tc_tiling_on_sc=False),
        scratch_types=[
            pltpu.VMEM((n_ch, _CH), jnp.int32),
            [pltpu.VMEM((_CH, D), f32) for _ in range(NB)],
            [pltpu.SemaphoreType.DMA for _ in range(NB)],
        ],
    )
    def gk(table_hbm, idx_hbm, out_hbm, idx_v, rows_v, sems):
        wid = lax.axis_index("s") * 2 + lax.axis_index("c")
        base = wid * per_w
        pltpu.sync_copy(idx_hbm.at[wid], idx_v)

        def body(g, c):
            i0 = g * NB
            copies = []
            for b in range(NB):
                copies.append(pltpu.async_copy(
                    table_hbm.at[idx_v.at[i0 + b]], rows_v[b], sems[b]))
            for b in range(NB):
                copies[b].wait()
                pltpu.sync_copy(rows_v[b],
                                out_hbm.at[pl.ds((base + (i0 + b) * _CH), _CH)])
            return c

        lax.fori_loop(0, n_ch // NB, body, 0)

    return gk(table, idxflat.reshape(_NW, n_ch, _CH))


def _edge_body(R, C, two_conv, g_ref, c_ref, w1_ref, ga1_ref, ba1_ref,
               w2_ref, ga2_ref, ba2_ref, out_ref):
    g = g_ref[0].reshape(R, K, C)
    c = c_ref[0]
    cb = jnp.broadcast_to(c[:, None, :], (R, K, C))
    fb = jnp.concatenate([(g - cb).astype(bf16), cb.astype(bf16)], axis=2)
    h = jnp.dot(fb.reshape(R * K, 2 * C), w1_ref[...],
                preferred_element_type=f32)
    h = _leaky(h * ga1_ref[0] + ba1_ref[0])
    if two_conv:
        h = jnp.dot(h.astype(bf16), w2_ref[...], preferred_element_type=f32)
        h = _leaky(h * ga2_ref[0] + ba2_ref[0])
    out_ref[0] = jnp.max(h.reshape(R, K, 64), axis=1)


def _edge(G, xt, w1, ga1, ba1, w2, ga2, ba2, two_conv, R=512):
    Bk, _, C = xt.shape
    return pl.pallas_call(
        functools.partial(_edge_body, R, C, two_conv),
        grid=(Bk, N // R),
        in_specs=[
            pl.BlockSpec((1, R * K, C), lambda b, r: (b, r, 0)),
            pl.BlockSpec((1, R, C), lambda b, r: (b, r, 0)),
            pl.BlockSpec((2 * C, 64), lambda b, r: (0, 0)),
            pl.BlockSpec((1, 64), lambda b, r: (0, 0)),
            pl.BlockSpec((1, 64), lambda b, r: (0, 0)),
            pl.BlockSpec((64, 64), lambda b, r: (0, 0)),
            pl.BlockSpec((1, 64), lambda b, r: (0, 0)),
            pl.BlockSpec((1, 64), lambda b, r: (0, 0)),
        ],
        out_specs=pl.BlockSpec((1, R, 64), lambda b, r: (b, r, 0)),
        out_shape=jax.ShapeDtypeStruct((Bk, N, 64), f32),
    )(G, xt, w1, ga1, ba1, w2, ga2, ba2)


def _tail_body(x1_ref, x2_ref, x3_ref, w6_ref, w7h_ref, w7x_ref, w8_ref,
               wc_ref, ga6_ref, ba6_ref, ga7_ref, ba7_ref, ga8_ref, ba8_ref,
               out_ref):
    xc = jnp.concatenate([x1_ref[0], x2_ref[0], x3_ref[0]], axis=1)
    xcb = xc.astype(bf16)
    h6 = _leaky(jnp.dot(xcb, w6_ref[...], preferred_element_type=f32)
                * ga6_ref[0] + ba6_ref[0])
    hmax = jnp.max(h6, axis=0, keepdims=True)
    t7 = jnp.dot(hmax.astype(bf16), w7h_ref[...], preferred_element_type=f32)
    h7 = _leaky((jnp.dot(xcb, w7x_ref[...], preferred_element_type=f32)
                 + t7) * ga7_ref[0] + ba7_ref[0])
    h8 = _leaky(jnp.dot(h7.astype(bf16), w8_ref[...],
                        preferred_element_type=f32)
                * ga8_ref[0] + ba8_ref[0])
    out_ref[0] = jnp.dot(h8.astype(bf16), wc_ref[...],
                         preferred_element_type=f32)


def _tail(x1t, x2t, x3t, w6, w7h, w7x, w8, wc, ga6, ba6, ga7, ba7, ga8, ba8):
    Bk = x1t.shape[0]
    cfull = lambda shape: pl.BlockSpec(shape, lambda b: tuple(0 for _ in shape))
    return pl.pallas_call(
        _tail_body,
        grid=(Bk,),
        in_specs=[
            pl.BlockSpec((1, N, 64), lambda b: (b, 0, 0)),
            pl.BlockSpec((1, N, 64), lambda b: (b, 0, 0)),
            pl.BlockSpec((1, N, 64), lambda b: (b, 0, 0)),
            cfull((192, 1024)),
            cfull((1024, 512)),
            cfull((192, 512)),
            cfull((512, 256)),
            cfull((256, 17)),
            cfull((1, 1024)), cfull((1, 1024)),
            cfull((1, 512)), cfull((1, 512)),
            cfull((1, 256)), cfull((1, 256)),
        ],
        out_specs=pl.BlockSpec((1, N, 17), lambda b: (b, 0, 0)),
        out_shape=jax.ShapeDtypeStruct((Bk, N, 17), f32),
    )(x1t, x2t, x3t, w6, w7h, w7x, w8, wc, ga6, ba6, ga7, ba7, ga8, ba8)


def kernel(x_in, W1, W2, W3, W4, W5, W6, W7, W8, Wcls,
           g1, b1, g2, b2, g3, b3, g4, b4, g5, b5, g6, b6, g7, b7, g8, b8):
    x = x_in[0].astype(f32)
    inv = 1.0 / jnp.sqrt(jnp.asarray(1.0 + 1e-5, f32))

    def bnp(g, b):
        return (g * inv).reshape(1, -1), b.reshape(1, -1)

    ga1, ba1 = bnp(g1, b1)
    ga2, ba2 = bnp(g2, b2)
    ga3, ba3 = bnp(g3, b3)
    ga4, ba4 = bnp(g4, b4)
    ga5, ba5 = bnp(g5, b5)
    ga6, ba6 = bnp(g6, b6)
    ga7, ba7 = bnp(g7, b7)
    ga8, ba8 = bnp(g8, b8)

    def edge_w1(W, C):
        Ctrue = W.shape[1] // 2
        w = jnp.zeros((2 * C, 64), f32)
        w = w.at[:Ctrue].set(W[:, :Ctrue].T)
        w = w.at[C:C + Ctrue].set(W[:, Ctrue:].T)
        return w.astype(bf16)

    w1e = edge_w1(W1, 16)
    w2e = W2.T.astype(bf16)
    w3e = edge_w1(W3, 64)
    w4e = W4.T.astype(bf16)
    w5e = edge_w1(W5, 64)
    w6e = W6.T.astype(bf16)
    w7he = W7[:, :1024].T.astype(bf16)
    w7xe = W7[:, 1024:].T.astype(bf16)
    w8e = W8.T.astype(bf16)
    wce = Wcls.T.astype(bf16)

    def half(xh):
        Bk = xh.shape[0]
        Mk, Tk = Bk * N, Bk * N * K
        xpad = jnp.pad(xh, ((0, 0), (0, 10), (0, 0)))
        xt = jnp.transpose(xpad, (0, 2, 1))
        idx1 = _knn(xt, xpad)
        G1 = _sc_gather(xt.reshape(Mk, 16), idx1.reshape(Tk))
        x1t = _edge(G1.reshape(Bk, N * K, 16), xt, w1e, ga1, ba1,
                    w2e, ga2, ba2, True)
        x1c = jnp.transpose(x1t, (0, 2, 1))
        idx2 = _knn(x1t, x1c)
        G2 = _sc_gather(x1t.reshape(Mk, 64), idx2.reshape(Tk))
        x2t = _edge(G2.reshape(Bk, N * K, 64), x1t, w3e, ga3, ba3,
                    w4e, ga4, ba4, True)
        x2c = jnp.transpose(x2t, (0, 2, 1))
        idx3 = _knn(x2t, x2c)
        G3 = _sc_gather(x2t.reshape(Mk, 64), idx3.reshape(Tk))
        x3t = _edge(G3.reshape(Bk, N * K, 64), x2t, w5e, ga5, ba5,
                    w4e, ga5, ba5, False)
        return _tail(x1t, x2t, x3t, w6e, w7he, w7xe, w8e, wce,
                     ga6, ba6, ga7, ba7, ga8, ba8)

    out = jnp.concatenate([half(x[:4]), half(x[4:])], axis=0)
    return jnp.transpose(out, (0, 2, 1))
